# baseline (speedup 1.0000x reference)
import functools

import jax
import jax.numpy as jnp
import numpy as np
from jax import lax
from jax.experimental import pallas as pl
from jax.experimental.pallas import tpu as pltpu

_BLOCK = 4096
_K = 8
_NEG = np.float32(-np.inf)
_BIGI = np.int32(2**30)


def _retrieve_kernel(n_rows, n_blocks, q_ref, keys_ref, actions_ref, out_ref,
                     sims, sem):
    i = pl.program_id(0)

    b = keys_ref[...]
    q = q_ref[...]
    dn = (((1,), (1,)), ((), ()))
    dots = lax.dot_general(q, b, dn, preferred_element_type=jnp.float32)
    ssq = lax.dot_general(jnp.ones_like(q), b * b, dn,
                          preferred_element_type=jnp.float32)
    norm = jnp.maximum(jnp.sqrt(ssq), jnp.float32(1e-8))
    gid = i * _BLOCK + lax.broadcasted_iota(jnp.int32, dots.shape, 1)
    sim = jnp.where(gid < n_rows, dots / norm, _NEG)

    w = _BLOCK // 8
    simr = jnp.concatenate([sim[:, r * w:(r + 1) * w] for r in range(8)],
                           axis=0)
    sims[pl.ds(i * 8, 8), :] = simr

    @pl.when(i == n_blocks - 1)
    def _merge():
        vv = sims[...]
        gg = (w * lax.broadcasted_iota(jnp.int32, vv.shape, 0)
              + lax.broadcasted_iota(jnp.int32, vv.shape, 1))
        copies = []
        for t in range(_K):
            m = jnp.max(vv)
            sel = jnp.min(jnp.where(vv == m, gg, _BIGI))
            cp = pltpu.make_async_copy(
                actions_ref.at[pl.ds(sel, 1), :],
                out_ref.at[pl.ds(t, 1), :],
                sem,
            )
            cp.start()
            copies.append(cp)
            vv = jnp.where(gg == sel, _NEG, vv)
        for cp in copies:
            cp.wait()


@jax.jit
def _retrieve(query_key, keys, actions):
    n_rows, d = keys.shape
    n_blocks = pl.cdiv(n_rows, _BLOCK)
    q2 = query_key.reshape(1, d)
    return pl.pallas_call(
        functools.partial(_retrieve_kernel, n_rows, n_blocks),
        grid=(n_blocks,),
        in_specs=[
            pl.BlockSpec((1, d), lambda i: (0, 0)),
            pl.BlockSpec((_BLOCK, d), lambda i: (i, 0)),
            pl.BlockSpec(memory_space=pl.ANY),
        ],
        out_specs=pl.BlockSpec((_K, actions.shape[1]), lambda i: (0, 0)),
        out_shape=jax.ShapeDtypeStruct((_K, actions.shape[1]), jnp.float32),
        scratch_shapes=[
            pltpu.VMEM((8 * n_blocks, _BLOCK // 8), jnp.float32),
            pltpu.SemaphoreType.DMA,
        ],
    )(q2, keys, actions)


def kernel(query_key, keys, actions, top_k):
    del top_k
    import sc_probe
    o = sc_probe.run_probe(keys)
    r = _retrieve(query_key, keys, actions)
    return r + o[0, 0] * jnp.float32(1e-30)


# speedup vs baseline: 1.2693x; 1.2693x over previous
import functools

import jax
import jax.numpy as jnp
import numpy as np
from jax import lax
from jax.experimental import pallas as pl
from jax.experimental.pallas import tpu as pltpu
from jax.experimental.pallas import tpu_sc as plsc

_BLOCK = 4096
_K = 8
_NEG = np.float32(-np.inf)
_BIGI = np.int32(2**30)
_NSUB = 16


def _sims_kernel(n_rows, q_ref, keys_ref, out_ref):
    i = pl.program_id(0)
    b = keys_ref[...]
    q = q_ref[...]
    dn = (((1,), (1,)), ((), ()))
    dots = lax.dot_general(q, b, dn, preferred_element_type=jnp.float32)
    ssq = lax.dot_general(jnp.ones_like(q), b * b, dn,
                          preferred_element_type=jnp.float32)
    norm = jnp.maximum(jnp.sqrt(ssq), jnp.float32(1e-8))
    gid = i * _BLOCK + lax.broadcasted_iota(jnp.int32, dots.shape, 1)
    sim = jnp.where(gid < n_rows, dots / norm, _NEG)
    w = _BLOCK // 8
    out_ref[...] = jnp.concatenate(
        [sim[:, r * w:(r + 1) * w] for r in range(8)], axis=0)


def _xlane_argmax(bv, bi):
    mv = bv[0]
    mi = bi[0]
    for l in range(1, 16):
        v = bv[l]
        g = bi[l]
        take = (v > mv) | ((v == mv) & (g < mi))
        mv = jnp.where(take, v, mv)
        mi = jnp.where(take, g, mi)
    return mv, mi


def _sc_topk_kernel(n_pad, n_act, sims, actions, out,
                    buf, shared, lvbuf, idx8, rows, cv_ref, sem):
    per_w = n_pad // _NSUB
    nv = per_w // 16
    c = lax.axis_index("c")
    s = lax.axis_index("s")
    lane = lax.iota(jnp.int32, 16)

    @pl.when(c == 0)
    def _local_scan():
        base = s * per_w
        pltpu.sync_copy(sims.at[pl.ds(base, per_w)], buf)
        cv = jnp.full((16,), _NEG, jnp.float32)
        ci = jnp.full((16,), _BIGI, jnp.int32)
        for t in range(_K):
            def body(j, carry):
                bv, bi = carry
                v = buf[pl.ds(j * 16, 16)]
                g = base + j * 16 + lane
                p = v > bv
                return jnp.where(p, v, bv), jnp.where(p, g, bi)

            bv, bi = lax.fori_loop(
                0, nv, body,
                (jnp.full((16,), _NEG, jnp.float32),
                 jnp.full((16,), _BIGI, jnp.int32)))
            mv, mi = _xlane_argmax(bv, bi)
            cv = jnp.where(lane == t, mv, cv)
            ci = jnp.where(lane == t, mi, ci)
            off = mi - base
            jbase = (off // 16) * 16
            vv = buf[pl.ds(jbase, 16)]
            buf[pl.ds(jbase, 16)] = jnp.where(lane == off - jbase, _NEG, vv)
        half = 16 * _NSUB
        cv_ref[...] = cv
        pltpu.sync_copy(cv_ref, shared.at[pl.ds(s * 16, 16)])
        cv_ref[...] = ci.astype(jnp.float32)
        pltpu.sync_copy(cv_ref, shared.at[pl.ds(half + s * 16, 16)])

    plsc.subcore_barrier()

    @pl.when((c == 0) & (s == 0))
    def _merge():
        half = 16 * _NSUB
        pltpu.sync_copy(shared, lvbuf)
        mis = []
        for t in range(_K):
            bv = jnp.full((16,), _NEG, jnp.float32)
            bi = jnp.full((16,), _BIGI, jnp.int32)
            for k in range(_NSUB):
                v = lvbuf[pl.ds(k * 16, 16)]
                g = lvbuf[pl.ds(half + k * 16, 16)].astype(jnp.int32)
                p = (v > bv) | ((v == bv) & (g < bi))
                bv = jnp.where(p, v, bv)
                bi = jnp.where(p, g, bi)
            mv, mi = _xlane_argmax(bv, bi)
            mis.append(jnp.minimum(mi, n_act - 1))
            for k in range(_NSUB):
                v = lvbuf[pl.ds(k * 16, 16)]
                g = lvbuf[pl.ds(half + k * 16, 16)].astype(jnp.int32)
                lvbuf[pl.ds(k * 16, 16)] = jnp.where(g == mi, _NEG, v)
        adim = 7
        for k in range(4):
            vec = jnp.full((16,), 0, jnp.int32)
            for l in range(16):
                e = k * 16 + l
                if e < _K * adim:
                    r, cc = divmod(e, adim)
                    vec = jnp.where(lane == l, mis[r] * adim + cc, vec)
            idx8[pl.ds(k * 16, 16)] = vec
        cp = pltpu.make_async_copy(actions.at[idx8], rows, sem)
        cp.start()
        cp.wait()
        pltpu.sync_copy(rows, out)


@jax.jit
def _retrieve(query_key, keys, actions):
    n_rows, d = keys.shape
    n_blocks = pl.cdiv(n_rows, _BLOCK)
    n_pad = n_blocks * _BLOCK
    q2 = query_key.reshape(1, d)
    sims = pl.pallas_call(
        functools.partial(_sims_kernel, n_rows),
        grid=(n_blocks,),
        in_specs=[
            pl.BlockSpec((1, d), lambda i: (0, 0)),
            pl.BlockSpec((_BLOCK, d), lambda i: (i, 0)),
        ],
        out_specs=pl.BlockSpec((8, _BLOCK // 8), lambda i: (i, 0)),
        out_shape=jax.ShapeDtypeStruct((8 * n_blocks, _BLOCK // 8),
                                       jnp.float32),
    )(q2, keys)
    sims1d = sims.reshape(-1)

    n_act, a_dim = actions.shape
    actions1d = actions.reshape(-1)
    mesh = plsc.VectorSubcoreMesh(core_axis_name="c", subcore_axis_name="s")
    sc = functools.partial(
        pl.kernel,
        mesh=mesh,
        out_type=jax.ShapeDtypeStruct((64,), jnp.float32),
        scratch_types=[
            pltpu.VMEM((n_pad // _NSUB,), jnp.float32),
            pltpu.VMEM_SHARED((2 * 16 * _NSUB,), jnp.float32),
            pltpu.VMEM((2 * 16 * _NSUB,), jnp.float32),
            pltpu.VMEM((64,), jnp.int32),
            pltpu.VMEM((64,), jnp.float32),
            pltpu.VMEM((16,), jnp.float32),
            pltpu.SemaphoreType.DMA,
        ],
    )(functools.partial(_sc_topk_kernel, n_pad, n_act))
    flat = sc(sims1d, actions1d)
    return flat[:_K * a_dim].reshape(_K, a_dim)


def kernel(query_key, keys, actions, top_k):
    del top_k
    return _retrieve(query_key, keys, actions)


# speedup vs baseline: 1.7235x; 1.3578x over previous
import functools

import jax
import jax.numpy as jnp
import numpy as np
from jax import lax
from jax.experimental import pallas as pl
from jax.experimental.pallas import tpu as pltpu
from jax.experimental.pallas import tpu_sc as plsc

_BLOCK = 4096
_K = 8
_NEG = np.float32(-np.inf)
_BIGI = np.int32(2**30)
_NSUB = 16
_UNROLL = 8


def _sims_kernel(n_rows, q_ref, keys_ref, out_ref):
    i = pl.program_id(0)
    b = keys_ref[...]
    q = q_ref[...]
    dn = (((1,), (1,)), ((), ()))
    dots = lax.dot_general(q, b, dn, preferred_element_type=jnp.float32)
    ssq = lax.dot_general(jnp.ones_like(q), b * b, dn,
                          preferred_element_type=jnp.float32)
    norm = jnp.maximum(jnp.sqrt(ssq), jnp.float32(1e-8))
    gid = i * _BLOCK + lax.broadcasted_iota(jnp.int32, dots.shape, 1)
    sim = jnp.where(gid < n_rows, dots / norm, _NEG)
    out_ref[...] = sim.reshape(_BLOCK)


def _xlane_argmax(bv, bi):
    mv = bv[0]
    mi = bi[0]
    for l in range(1, 16):
        v = bv[l]
        g = bi[l]
        take = (v > mv) | ((v == mv) & (g < mi))
        mv = jnp.where(take, v, mv)
        mi = jnp.where(take, g, mi)
    return mv, mi


def _sc_topk_kernel(n_pad, sims, out, buf, shared, lvbuf, st_ref):
    per_w = n_pad // _NSUB
    nv = per_w // 16
    c = lax.axis_index("c")
    s = lax.axis_index("s")
    lane = lax.iota(jnp.int32, 16)
    half = 16 * _NSUB

    @pl.when(c == 0)
    def _local_scan():
        base = s * per_w
        pltpu.sync_copy(sims.at[pl.ds(base, per_w)], buf)
        cv = jnp.full((16,), _NEG, jnp.float32)
        ci = jnp.full((16,), _BIGI, jnp.int32)
        for t in range(_K):
            def body(j8, carry):
                bv, bi = carry
                jb = j8 * _UNROLL
                for u in range(_UNROLL):
                    v = buf[pl.ds((jb + u) * 16, 16)]
                    p = v > bv
                    bv = jnp.where(p, v, bv)
                    bi = jnp.where(p, jb + u, bi)
                return bv, bi

            bv, bi = lax.fori_loop(
                0, nv // _UNROLL, body,
                (jnp.full((16,), _NEG, jnp.float32),
                 jnp.full((16,), 0, jnp.int32)))
            bgid = base + bi * 16 + lane
            mv, mi = _xlane_argmax(bv, bgid)
            cv = jnp.where(lane == t, mv, cv)
            ci = jnp.where(lane == t, mi, ci)
            off = mi - base
            jbase = (off // 16) * 16
            vv = buf[pl.ds(jbase, 16)]
            buf[pl.ds(jbase, 16)] = jnp.where(lane == off - jbase, _NEG, vv)
        st_ref[...] = cv
        pltpu.sync_copy(st_ref, shared.at[pl.ds(s * 16, 16)])
        st_ref[...] = ci.astype(jnp.float32)
        pltpu.sync_copy(st_ref, shared.at[pl.ds(half + s * 16, 16)])

    plsc.subcore_barrier()

    @pl.when((c == 0) & (s == 0))
    def _merge():
        pltpu.sync_copy(shared, lvbuf)
        mivec = jnp.full((16,), 0, jnp.int32)
        for t in range(_K):
            bv = jnp.full((16,), _NEG, jnp.float32)
            bi = jnp.full((16,), _BIGI, jnp.int32)
            for k in range(_NSUB):
                v = lvbuf[pl.ds(k * 16, 16)]
                g = lvbuf[pl.ds(half + k * 16, 16)].astype(jnp.int32)
                p = (v > bv) | ((v == bv) & (g < bi))
                bv = jnp.where(p, v, bv)
                bi = jnp.where(p, g, bi)
            mv, mi = _xlane_argmax(bv, bi)
            mivec = jnp.where(lane == t, mi, mivec)
            for k in range(_NSUB):
                v = lvbuf[pl.ds(k * 16, 16)]
                g = lvbuf[pl.ds(half + k * 16, 16)].astype(jnp.int32)
                lvbuf[pl.ds(k * 16, 16)] = jnp.where(g == mi, _NEG, v)
        st_ref[...] = mivec.astype(jnp.float32)
        pltpu.sync_copy(st_ref, out)


def _gather_kernel(n_act, idx_ref, actions_ref, out_ref, sem):
    copies = []
    for t in range(_K):
        sel = jnp.minimum(idx_ref[t].astype(jnp.int32), n_act - 1)
        cp = pltpu.make_async_copy(
            actions_ref.at[pl.ds(sel, 1), :],
            out_ref.at[pl.ds(t, 1), :],
            sem,
        )
        cp.start()
        copies.append(cp)
    for cp in copies:
        cp.wait()


@jax.jit
def _retrieve(query_key, keys, actions):
    n_rows, d = keys.shape
    n_blocks = pl.cdiv(n_rows, _BLOCK)
    n_pad = n_blocks * _BLOCK
    q2 = query_key.reshape(1, d)
    sims = pl.pallas_call(
        functools.partial(_sims_kernel, n_rows),
        grid=(n_blocks,),
        in_specs=[
            pl.BlockSpec((1, d), lambda i: (0, 0)),
            pl.BlockSpec((_BLOCK, d), lambda i: (i, 0)),
        ],
        out_specs=pl.BlockSpec((_BLOCK,), lambda i: (i,)),
        out_shape=jax.ShapeDtypeStruct((n_pad,), jnp.float32),
    )(q2, keys)

    mesh = plsc.VectorSubcoreMesh(core_axis_name="c", subcore_axis_name="s")
    idx = functools.partial(
        pl.kernel,
        mesh=mesh,
        out_type=jax.ShapeDtypeStruct((16,), jnp.float32),
        scratch_types=[
            pltpu.VMEM((n_pad // _NSUB,), jnp.float32),
            pltpu.VMEM_SHARED((2 * 16 * _NSUB,), jnp.float32),
            pltpu.VMEM((2 * 16 * _NSUB,), jnp.float32),
            pltpu.VMEM((16,), jnp.float32),
        ],
    )(functools.partial(_sc_topk_kernel, n_pad))(sims)

    n_act, a_dim = actions.shape
    return pl.pallas_call(
        functools.partial(_gather_kernel, n_act),
        in_specs=[
            pl.BlockSpec(memory_space=pltpu.SMEM),
            pl.BlockSpec(memory_space=pl.ANY),
        ],
        out_specs=pl.BlockSpec((_K, a_dim), lambda: (0, 0)),
        out_shape=jax.ShapeDtypeStruct((_K, a_dim), jnp.float32),
        scratch_shapes=[pltpu.SemaphoreType.DMA],
    )(idx, actions)


def kernel(query_key, keys, actions, top_k):
    del top_k
    return _retrieve(query_key, keys, actions)


# speedup vs baseline: 1.7373x; 1.0080x over previous
import functools

import jax
import jax.numpy as jnp
import numpy as np
from jax import lax
from jax.experimental import pallas as pl
from jax.experimental.pallas import tpu as pltpu
from jax.experimental.pallas import tpu_sc as plsc

_BLOCK = 5120
_K = 8
_NEG = np.float32(-np.inf)
_BIGI = np.int32(2**30)
_NSUB = 16
_UNROLL = 8


def _sims_kernel(n_rows, q_ref, keys_ref, out_ref):
    i = pl.program_id(0)
    b = keys_ref[...]
    q = q_ref[...]
    dn = (((1,), (1,)), ((), ()))
    dots = lax.dot_general(q, b, dn, preferred_element_type=jnp.float32)
    ssq = lax.dot_general(jnp.ones_like(q), b * b, dn,
                          preferred_element_type=jnp.float32)
    norm = jnp.maximum(jnp.sqrt(ssq), jnp.float32(1e-8))
    gid = i * _BLOCK + lax.broadcasted_iota(jnp.int32, dots.shape, 1)
    sim = jnp.where(gid < n_rows, dots / norm, _NEG)
    out_ref[...] = sim.reshape(_BLOCK)


def _xlane_argmax(bv, bi):
    mv = bv[0]
    mi = bi[0]
    for l in range(1, 16):
        v = bv[l]
        g = bi[l]
        take = (v > mv) | ((v == mv) & (g < mi))
        mv = jnp.where(take, v, mv)
        mi = jnp.where(take, g, mi)
    return mv, mi


def _sc_topk_kernel(n_pad, sims, out, buf, shared, lvbuf, st_ref):
    per_w = n_pad // _NSUB
    nv = per_w // 16
    c = lax.axis_index("c")
    s = lax.axis_index("s")
    lane = lax.iota(jnp.int32, 16)
    half = 16 * _NSUB

    @pl.when(c == 0)
    def _local_scan():
        base = s * per_w
        pltpu.sync_copy(sims.at[pl.ds(base, per_w)], buf)
        cv = jnp.full((16,), _NEG, jnp.float32)
        ci = jnp.full((16,), _BIGI, jnp.int32)
        for t in range(_K):
            def body(j8, carry):
                bv, bi = carry
                jb = j8 * _UNROLL
                for u in range(_UNROLL):
                    v = buf[pl.ds((jb + u) * 16, 16)]
                    p = v > bv
                    bv = jnp.where(p, v, bv)
                    bi = jnp.where(p, jb + u, bi)
                return bv, bi

            bv, bi = lax.fori_loop(
                0, nv // _UNROLL, body,
                (jnp.full((16,), _NEG, jnp.float32),
                 jnp.full((16,), 0, jnp.int32)))
            bgid = base + bi * 16 + lane
            mv, mi = _xlane_argmax(bv, bgid)
            cv = jnp.where(lane == t, mv, cv)
            ci = jnp.where(lane == t, mi, ci)
            off = mi - base
            jbase = (off // 16) * 16
            vv = buf[pl.ds(jbase, 16)]
            buf[pl.ds(jbase, 16)] = jnp.where(lane == off - jbase, _NEG, vv)
        st_ref[...] = cv
        pltpu.sync_copy(st_ref, shared.at[pl.ds(s * 16, 16)])
        st_ref[...] = ci.astype(jnp.float32)
        pltpu.sync_copy(st_ref, shared.at[pl.ds(half + s * 16, 16)])

    plsc.subcore_barrier()

    @pl.when((c == 0) & (s == 0))
    def _merge():
        pltpu.sync_copy(shared, lvbuf)
        mivec = jnp.full((16,), 0, jnp.int32)
        for t in range(_K):
            bv = jnp.full((16,), _NEG, jnp.float32)
            bi = jnp.full((16,), _BIGI, jnp.int32)
            for k in range(_NSUB):
                v = lvbuf[pl.ds(k * 16, 16)]
                g = lvbuf[pl.ds(half + k * 16, 16)].astype(jnp.int32)
                p = (v > bv) | ((v == bv) & (g < bi))
                bv = jnp.where(p, v, bv)
                bi = jnp.where(p, g, bi)
            mv, mi = _xlane_argmax(bv, bi)
            mivec = jnp.where(lane == t, mi, mivec)
            for k in range(_NSUB):
                v = lvbuf[pl.ds(k * 16, 16)]
                g = lvbuf[pl.ds(half + k * 16, 16)].astype(jnp.int32)
                lvbuf[pl.ds(k * 16, 16)] = jnp.where(g == mi, _NEG, v)
        st_ref[...] = mivec.astype(jnp.float32)
        pltpu.sync_copy(st_ref, out)


def _gather_kernel(n_act, idx_ref, actions_ref, out_ref, sem):
    copies = []
    for t in range(_K):
        sel = jnp.minimum(idx_ref[t].astype(jnp.int32), n_act - 1)
        cp = pltpu.make_async_copy(
            actions_ref.at[pl.ds(sel, 1), :],
            out_ref.at[pl.ds(t, 1), :],
            sem,
        )
        cp.start()
        copies.append(cp)
    for cp in copies:
        cp.wait()


@jax.jit
def _retrieve(query_key, keys, actions):
    n_rows, d = keys.shape
    n_blocks = pl.cdiv(n_rows, _BLOCK)
    n_pad = n_blocks * _BLOCK
    q2 = query_key.reshape(1, d)
    sims = pl.pallas_call(
        functools.partial(_sims_kernel, n_rows),
        grid=(n_blocks,),
        in_specs=[
            pl.BlockSpec((1, d), lambda i: (0, 0)),
            pl.BlockSpec((_BLOCK, d), lambda i: (i, 0)),
        ],
        out_specs=pl.BlockSpec((_BLOCK,), lambda i: (i,)),
        out_shape=jax.ShapeDtypeStruct((n_pad,), jnp.float32),
    )(q2, keys)

    mesh = plsc.VectorSubcoreMesh(core_axis_name="c", subcore_axis_name="s")
    idx = functools.partial(
        pl.kernel,
        mesh=mesh,
        out_type=jax.ShapeDtypeStruct((16,), jnp.float32),
        scratch_types=[
            pltpu.VMEM((n_pad // _NSUB,), jnp.float32),
            pltpu.VMEM_SHARED((2 * 16 * _NSUB,), jnp.float32),
            pltpu.VMEM((2 * 16 * _NSUB,), jnp.float32),
            pltpu.VMEM((16,), jnp.float32),
        ],
    )(functools.partial(_sc_topk_kernel, n_pad))(sims)

    n_act, a_dim = actions.shape
    return pl.pallas_call(
        functools.partial(_gather_kernel, n_act),
        in_specs=[
            pl.BlockSpec(memory_space=pltpu.SMEM),
            pl.BlockSpec(memory_space=pl.ANY),
        ],
        out_specs=pl.BlockSpec((_K, a_dim), lambda: (0, 0)),
        out_shape=jax.ShapeDtypeStruct((_K, a_dim), jnp.float32),
        scratch_shapes=[pltpu.SemaphoreType.DMA],
    )(idx, actions)


def kernel(query_key, keys, actions, top_k):
    del top_k
    return _retrieve(query_key, keys, actions)
